# baseline (speedup 1.0000x reference)
import functools

import jax
import jax.numpy as jnp
from jax import lax
from jax.experimental import pallas as pl
from jax.experimental.pallas import tpu as pltpu
from jax.experimental.pallas import tpu_sc as plsc

_NUM_WORKERS = 32
_NBUF = 2
_CHUNK = 256
_G = 128


@functools.partial(jax.jit, static_argnums=(2, 3, 4))
def _emb5(ids_t, weights, n_s, n_b, d):
  n_jh, n_bh = d // 8, n_b // 128
  bcols = n_b // _NUM_WORKERS
  assert bcols == _NBUF * _CHUNK
  mesh = plsc.VectorSubcoreMesh(core_axis_name="c", subcore_axis_name="s")

  @functools.partial(
      pl.kernel,
      mesh=mesh,
      out_type=jax.ShapeDtypeStruct((n_s, n_jh, n_bh, 8, 128), jnp.float32),
      scratch_types=(
          [pltpu.VMEM((_CHUNK,), jnp.int32) for _ in range(_NBUF)]
          + [pltpu.VMEM((_CHUNK, d), jnp.float32) for _ in range(_NBUF)]
          + [pltpu.VMEM((n_jh, _CHUNK // 128, 8, 128), jnp.float32)
             for _ in range(_NBUF)]
          + [pltpu.SemaphoreType.DMA] * (3 * _NBUF)
      ),
      compiler_params=pltpu.CompilerParams(
          use_tc_tiling_on_sc=False, needs_layout_passes=False),
  )
  def emb_kernel(ids_hbm, table_hbm, out_hbm, *scr):
    idx_v = scr[0:_NBUF]
    rows_v = scr[_NBUF:2 * _NBUF]
    ob_v = scr[2 * _NBUF:3 * _NBUF]
    sem_i = scr[3 * _NBUF:4 * _NBUF]
    sem_g = scr[4 * _NBUF:5 * _NBUF]
    sem_o = scr[5 * _NBUF:6 * _NBUF]

    wid = lax.axis_index("s") * 2 + lax.axis_index("c")
    wb0 = wid * bcols
    wbh0 = wid * (bcols // 128)
    nbh_c = _CHUNK // 128

    lane = jnp.arange(16, dtype=jnp.int32)
    i_jh = [(lane + j0) >> 3 for j0 in range(0, d, 16)]
    i_jl = [(lane + j0) & 7 for j0 in range(0, d, 16)]

    def idx_desc(s, b):
      return pltpu.make_async_copy(
          ids_hbm.at[s, pl.ds(wb0 + b * _CHUNK, _CHUNK)],
          idx_v[b], sem_i[b])

    def out_desc(s, b):
      return pltpu.make_async_copy(
          ob_v[b],
          out_hbm.at[s, :, pl.ds(wbh0 + b * nbh_c, nbh_c)],
          sem_o[b])

    for b in range(_NBUF):
      idx_desc(0, b).start()

    @pl.loop(0, n_s)
    def s_body(s):
      for b in range(_NBUF):
        @pl.when(s > 0)
        def _(b=b):
          out_desc(s - 1, b).wait()
      gd = []
      for b in range(_NBUF):
        idx_desc(s, b).wait()
        for g in range(_CHUNK // _G):
          gd.append(pltpu.async_copy(
              table_hbm.at[idx_v[b].at[pl.ds(g * _G, _G)]],
              rows_v[b].at[pl.ds(g * _G, _G)],
              sem_g[b]))
      k = 0
      for b in range(_NBUF):
        for _ in range(_CHUNK // _G):
          gd[k].wait()
          k += 1
        rows, ob = rows_v[b], ob_v[b]

        @pl.loop(0, _CHUNK, unroll=4)
        def tok_body(tok, rows=rows, ob=ob):
          i_bh = jnp.broadcast_to(tok >> 7, (16,)).astype(jnp.int32)
          i_bl = jnp.broadcast_to(tok & 127, (16,)).astype(jnp.int32)
          for jj, j0 in enumerate(range(0, d, 16)):
            vals = rows[tok, pl.ds(j0, 16)]
            plsc.store_scatter(ob, [i_jh[jj], i_bh, i_jl[jj], i_bl], vals)

        out_desc(s, b).start()
        @pl.when(s < n_s - 1)
        def _(b=b):
          idx_desc(s + 1, b).start()

    for b in range(_NBUF):
      out_desc(n_s - 1, b).wait()

  return emb_kernel(ids_t, weights)


def kernel(token_ids, weights):
  bsz, seq = token_ids.shape
  d_model = weights.shape[1]
  ids_t = token_ids.T.astype(jnp.int32)
  out5 = _emb5(ids_t, weights, seq, bsz, d_model)
  return out5.transpose(2, 4, 0, 1, 3).reshape(bsz, seq, d_model)


# speedup vs baseline: 1.0525x; 1.0525x over previous
import functools

import jax
import jax.numpy as jnp
from jax import lax
from jax.experimental import pallas as pl
from jax.experimental.pallas import tpu as pltpu
from jax.experimental.pallas import tpu_sc as plsc

_NUM_WORKERS = 32
_NBUF = 2
_CHUNK = 256
_G = 128


@functools.partial(jax.jit, static_argnums=(2, 3, 4))
def _emb5(ids_t, weights, n_s, n_b, d):
  n_jh = d // 8
  bcols = n_b // _NUM_WORKERS
  assert bcols == _NBUF * _CHUNK
  nbh_c = _CHUNK // 128
  obw = _CHUNK * d
  jhw = nbh_c * 1024
  mesh = plsc.VectorSubcoreMesh(core_axis_name="c", subcore_axis_name="s")

  @functools.partial(
      pl.kernel,
      mesh=mesh,
      out_type=jax.ShapeDtypeStruct((n_s, n_jh, n_b // 128, 8, 128),
                                    jnp.float32),
      scratch_types=(
          [pltpu.VMEM((_CHUNK,), jnp.int32) for _ in range(2 * _NBUF)]
          + [pltpu.VMEM((_CHUNK, d), jnp.float32) for _ in range(2 * _NBUF)]
          + [pltpu.VMEM((n_jh, nbh_c, 8, 128), jnp.float32)
             for _ in range(_NBUF)]
          + [pltpu.SemaphoreType.DMA] * (4 * _NBUF + _NBUF)
      ),
      compiler_params=pltpu.CompilerParams(
          use_tc_tiling_on_sc=False, needs_layout_passes=False),
  )
  def emb_kernel(ids_hbm, table_hbm, out_hbm, *scr):
    idx_v = [scr[0:_NBUF], scr[_NBUF:2 * _NBUF]]
    rows_v = [scr[2 * _NBUF:3 * _NBUF], scr[3 * _NBUF:4 * _NBUF]]
    ob_v = scr[4 * _NBUF:5 * _NBUF]
    sem_i = [scr[5 * _NBUF:6 * _NBUF], scr[6 * _NBUF:7 * _NBUF]]
    sem_g = [scr[7 * _NBUF:8 * _NBUF], scr[8 * _NBUF:9 * _NBUF]]
    sem_o = scr[9 * _NBUF:10 * _NBUF]

    wid = lax.axis_index("s") * 2 + lax.axis_index("c")
    wb0 = wid * bcols
    wbh0 = wid * (bcols // 128)

    lane = jnp.arange(16, dtype=jnp.int32)
    i_jh = [(lane + j0) >> 3 for j0 in range(0, d, 16)]
    i_jl = [(lane + j0) & 7 for j0 in range(0, d, 16)]

    def idx_desc(s, p, b):
      return pltpu.make_async_copy(
          ids_hbm.at[s, pl.ds(wb0 + b * _CHUNK, _CHUNK)],
          idx_v[p][b], sem_i[p][b])

    def gather_descs(p, b):
      return [pltpu.make_async_copy(
          table_hbm.at[idx_v[p][b].at[pl.ds(g * _G, _G)]],
          rows_v[p][b].at[pl.ds(g * _G, _G)],
          sem_g[p][b]) for g in range(_CHUNK // _G)]

    def out_descs(s, b):
      return [pltpu.make_async_copy(
          ob_v[b].at[jh],
          out_hbm.at[s, jh, pl.ds(wbh0 + b * nbh_c, nbh_c)],
          sem_o[b]) for jh in range(n_jh)]

    def transpose_chunk(rows, ob):
      @pl.loop(0, _CHUNK, unroll=8)
      def tok_body(tok):
        i_bh = jnp.broadcast_to(tok >> 7, (16,))
        i_bl = jnp.broadcast_to(tok & 127, (16,))
        for jj, j0 in enumerate(range(0, d, 16)):
          vals = rows[tok, pl.ds(j0, 16)]
          plsc.store_scatter(ob, [i_jh[jj], i_bh, i_jl[jj], i_bl], vals)

    for b in range(_NBUF):
      idx_desc(0, 0, b).start()
      idx_desc(1, 1, b).start()

    @pl.loop(0, n_s + 2, step=2)
    def round_body(r0):
      for dp in range(2):
        rr = r0 + dp
        p = dp
        @pl.when(rr < n_s)
        def _(rr=rr, p=p):
          for b in range(_NBUF):
            idx_desc(rr, p, b).wait()
            for dsc in gather_descs(p, b):
              dsc.start()
        @pl.when(jnp.logical_and(rr >= 1, rr <= n_s))
        def _(rr=rr, p=p):
          for b in range(_NBUF):
            for dsc in gather_descs(1 - p, b):
              dsc.wait()
            @pl.when(rr >= 2)
            def _(rr=rr, b=b):
              for dsc in out_descs(rr - 2, b):
                dsc.wait()
            transpose_chunk(rows_v[1 - p][b], ob_v[b])
            for dsc in out_descs(rr - 1, b):
              dsc.start()
            @pl.when(rr + 1 < n_s)
            def _(rr=rr, p=p, b=b):
              idx_desc(rr + 1, 1 - p, b).start()

    for b in range(_NBUF):
      for dsc in out_descs(n_s - 1, b):
        dsc.wait()

  return emb_kernel(ids_t, weights)


def kernel(token_ids, weights):
  bsz, seq = token_ids.shape
  d_model = weights.shape[1]
  ids_t = token_ids.T.astype(jnp.int32)
  out5 = _emb5(ids_t, weights, seq, bsz, d_model)
  return out5.transpose(2, 4, 0, 1, 3).reshape(bsz, seq, d_model)


# speedup vs baseline: 1.2774x; 1.2137x over previous
import functools

import jax
import jax.numpy as jnp
from jax import lax
from jax.experimental import pallas as pl
from jax.experimental.pallas import tpu as pltpu
from jax.experimental.pallas import tpu_sc as plsc

_NUM_WORKERS = 32
_NBUF = 2
_CHUNK = 256
_G = 128


@functools.partial(jax.jit, static_argnums=(2, 3, 4))
def _emb5(ids_t, weights, n_s, n_b, d):
  n_jh = d // 8
  bcols = n_b // _NUM_WORKERS
  assert bcols == _NBUF * _CHUNK
  nbh_c = _CHUNK // 128
  obw = _CHUNK * d
  jhw = nbh_c * 1024
  mesh = plsc.VectorSubcoreMesh(core_axis_name="c", subcore_axis_name="s")

  @functools.partial(
      pl.kernel,
      mesh=mesh,
      out_type=jax.ShapeDtypeStruct((n_s, n_jh, n_b // 128, 8, 128),
                                    jnp.float32),
      scratch_types=(
          [pltpu.VMEM((_CHUNK,), jnp.int32) for _ in range(2 * _NBUF)]
          + [pltpu.VMEM((_CHUNK, d), jnp.float32) for _ in range(2 * _NBUF)]
          + [pltpu.VMEM((n_jh, nbh_c, 8, 128), jnp.float32)
             for _ in range(_NBUF)]
          + [pltpu.SemaphoreType.DMA] * (4 * _NBUF + _NBUF)
      ),
      compiler_params=pltpu.CompilerParams(
          use_tc_tiling_on_sc=False, needs_layout_passes=False),
  )
  def emb_kernel(ids_hbm, table_hbm, out_hbm, *scr):
    idx_v = [scr[0:_NBUF], scr[_NBUF:2 * _NBUF]]
    rows_v = [scr[2 * _NBUF:3 * _NBUF], scr[3 * _NBUF:4 * _NBUF]]
    ob_v = scr[4 * _NBUF:5 * _NBUF]
    sem_i = [scr[5 * _NBUF:6 * _NBUF], scr[6 * _NBUF:7 * _NBUF]]
    sem_g = [scr[7 * _NBUF:8 * _NBUF], scr[8 * _NBUF:9 * _NBUF]]
    sem_o = scr[9 * _NBUF:10 * _NBUF]

    wid = lax.axis_index("s") * 2 + lax.axis_index("c")
    wb0 = wid * bcols
    wbh0 = wid * (bcols // 128)

    lane = jnp.arange(16, dtype=jnp.int32)
    i_jh = [(lane + j0) >> 3 for j0 in range(0, d, 16)]
    i_jl = [(lane + j0) & 7 for j0 in range(0, d, 16)]

    def idx_desc(s, p, b):
      return pltpu.make_async_copy(
          ids_hbm.at[s, pl.ds(wb0 + b * _CHUNK, _CHUNK)],
          idx_v[p][b], sem_i[p][b])

    def gather_descs(p, b):
      return [pltpu.make_async_copy(
          table_hbm.at[idx_v[p][b].at[pl.ds(g * _G, _G)]],
          rows_v[p][b].at[pl.ds(g * _G, _G)],
          sem_g[p][b]) for g in range(_CHUNK // _G)]

    def out_descs(s, b):
      return [pltpu.make_async_copy(
          ob_v[b].at[jh],
          out_hbm.at[s, jh, pl.ds(wbh0 + b * nbh_c, nbh_c)],
          sem_o[b]) for jh in range(n_jh)]

    def transpose_chunk(rows, ob):
      @plsc.parallel_loop(0, _CHUNK, unroll=8)
      def tok_body(tok):
        i_bh = jnp.broadcast_to(tok >> 7, (16,))
        i_bl = jnp.broadcast_to(tok & 127, (16,))
        for jj, j0 in enumerate(range(0, d, 16)):
          vals = rows[tok, pl.ds(j0, 16)]
          plsc.store_scatter(ob, [i_jh[jj], i_bh, i_jl[jj], i_bl], vals)

    for b in range(_NBUF):
      idx_desc(0, 0, b).start()
      idx_desc(1, 1, b).start()

    @pl.loop(0, n_s + 2, step=2)
    def round_body(r0):
      for dp in range(2):
        rr = r0 + dp
        p = dp
        @pl.when(rr < n_s)
        def _(rr=rr, p=p):
          for b in range(_NBUF):
            idx_desc(rr, p, b).wait()
            for dsc in gather_descs(p, b):
              dsc.start()
        @pl.when(jnp.logical_and(rr >= 1, rr <= n_s))
        def _(rr=rr, p=p):
          for b in range(_NBUF):
            for dsc in gather_descs(1 - p, b):
              dsc.wait()
            @pl.when(rr >= 2)
            def _(rr=rr, b=b):
              for dsc in out_descs(rr - 2, b):
                dsc.wait()
            transpose_chunk(rows_v[1 - p][b], ob_v[b])
            for dsc in out_descs(rr - 1, b):
              dsc.start()
            @pl.when(rr + 1 < n_s)
            def _(rr=rr, p=p, b=b):
              idx_desc(rr + 1, 1 - p, b).start()

    for b in range(_NBUF):
      for dsc in out_descs(n_s - 1, b):
        dsc.wait()

  return emb_kernel(ids_t, weights)


def kernel(token_ids, weights):
  bsz, seq = token_ids.shape
  d_model = weights.shape[1]
  ids_t = token_ids.T.astype(jnp.int32)
  out5 = _emb5(ids_t, weights, seq, bsz, d_model)
  return out5.transpose(2, 4, 0, 1, 3).reshape(bsz, seq, d_model)


# speedup vs baseline: 1.2804x; 1.0023x over previous
import functools

import jax
import jax.numpy as jnp
from jax import lax
from jax.experimental import pallas as pl
from jax.experimental.pallas import tpu as pltpu
from jax.experimental.pallas import tpu_sc as plsc

_NUM_WORKERS = 32
_NBUF = 2
_CHUNK = 256
_G = 128


@functools.partial(jax.jit, static_argnums=(2, 3, 4))
def _emb5(ids_t, weights, n_s, n_b, d):
  n_jh = d // 8
  bcols = n_b // _NUM_WORKERS
  assert bcols == _NBUF * _CHUNK
  nbh_c = _CHUNK // 128
  obw = _CHUNK * d
  jhw = nbh_c * 1024
  mesh = plsc.VectorSubcoreMesh(core_axis_name="c", subcore_axis_name="s")

  @functools.partial(
      pl.kernel,
      mesh=mesh,
      out_type=jax.ShapeDtypeStruct((n_s, n_jh, n_b // 128, 8, 128),
                                    jnp.float32),
      scratch_types=(
          [pltpu.VMEM((_CHUNK,), jnp.int32) for _ in range(2 * _NBUF)]
          + [pltpu.VMEM((_CHUNK, d), jnp.float32) for _ in range(2 * _NBUF)]
          + [pltpu.VMEM((n_jh, nbh_c, 8, 128), jnp.float32)
             for _ in range(_NBUF)]
          + [pltpu.SemaphoreType.DMA] * (4 * _NBUF + _NBUF)
      ),
      compiler_params=pltpu.CompilerParams(
          use_tc_tiling_on_sc=False, needs_layout_passes=False),
  )
  def emb_kernel(ids_hbm, table_hbm, out_hbm, *scr):
    idx_v = [scr[0:_NBUF], scr[_NBUF:2 * _NBUF]]
    rows_v = [scr[2 * _NBUF:3 * _NBUF], scr[3 * _NBUF:4 * _NBUF]]
    ob_v = scr[4 * _NBUF:5 * _NBUF]
    sem_i = [scr[5 * _NBUF:6 * _NBUF], scr[6 * _NBUF:7 * _NBUF]]
    sem_g = [scr[7 * _NBUF:8 * _NBUF], scr[8 * _NBUF:9 * _NBUF]]
    sem_o = scr[9 * _NBUF:10 * _NBUF]

    wid = lax.axis_index("s") * 2 + lax.axis_index("c")
    wb0 = wid * bcols
    wbh0 = wid * (bcols // 128)

    lane = jnp.arange(16, dtype=jnp.int32)
    i_jh = [(lane + j0) >> 3 for j0 in range(0, d, 16)]
    i_jl = [(lane + j0) & 7 for j0 in range(0, d, 16)]

    def idx_desc(s, p, b):
      return pltpu.make_async_copy(
          ids_hbm.at[s, pl.ds(wb0 + b * _CHUNK, _CHUNK)],
          idx_v[p][b], sem_i[p][b])

    def gather_descs(p, b):
      return [pltpu.make_async_copy(
          table_hbm.at[idx_v[p][b].at[pl.ds(g * _G, _G)]],
          rows_v[p][b].at[pl.ds(g * _G, _G)],
          sem_g[p][b]) for g in range(_CHUNK // _G)]

    def out_descs(s, b):
      return [pltpu.make_async_copy(
          ob_v[b],
          out_hbm.at[s, :, pl.ds(wbh0 + b * nbh_c, nbh_c)],
          sem_o[b])]

    def transpose_chunk(rows, ob):
      @plsc.parallel_loop(0, _CHUNK, unroll=8)
      def tok_body(tok):
        i_bh = jnp.broadcast_to(tok >> 7, (16,))
        i_bl = jnp.broadcast_to(tok & 127, (16,))
        for jj, j0 in enumerate(range(0, d, 16)):
          vals = rows[tok, pl.ds(j0, 16)]
          plsc.store_scatter(ob, [i_jh[jj], i_bh, i_jl[jj], i_bl], vals)

    for b in range(_NBUF):
      idx_desc(0, 0, b).start()
      idx_desc(1, 1, b).start()

    @pl.loop(0, n_s + 2, step=2)
    def round_body(r0):
      for dp in range(2):
        rr = r0 + dp
        p = dp
        @pl.when(rr < n_s)
        def _(rr=rr, p=p):
          for b in range(_NBUF):
            idx_desc(rr, p, b).wait()
            for dsc in gather_descs(p, b):
              dsc.start()
        @pl.when(jnp.logical_and(rr >= 1, rr <= n_s))
        def _(rr=rr, p=p):
          for b in range(_NBUF):
            for dsc in gather_descs(1 - p, b):
              dsc.wait()
            @pl.when(rr >= 2)
            def _(rr=rr, b=b):
              for dsc in out_descs(rr - 2, b):
                dsc.wait()
            transpose_chunk(rows_v[1 - p][b], ob_v[b])
            for dsc in out_descs(rr - 1, b):
              dsc.start()
            @pl.when(rr + 1 < n_s)
            def _(rr=rr, p=p, b=b):
              idx_desc(rr + 1, 1 - p, b).start()

    for b in range(_NBUF):
      for dsc in out_descs(n_s - 1, b):
        dsc.wait()

  return emb_kernel(ids_t, weights)


def kernel(token_ids, weights):
  bsz, seq = token_ids.shape
  d_model = weights.shape[1]
  ids_t = token_ids.T.astype(jnp.int32)
  out5 = _emb5(ids_t, weights, seq, bsz, d_model)
  return out5.transpose(2, 4, 0, 1, 3).reshape(bsz, seq, d_model)


# speedup vs baseline: 1.2832x; 1.0022x over previous
import functools

import jax
import jax.numpy as jnp
from jax import lax
from jax.experimental import pallas as pl
from jax.experimental.pallas import tpu as pltpu
from jax.experimental.pallas import tpu_sc as plsc

_NUM_WORKERS = 32
_NBUF = 4
_CHUNK = 128
_G = 128


@functools.partial(jax.jit, static_argnums=(2, 3, 4))
def _emb5(ids_t, weights, n_s, n_b, d):
  n_jh = d // 8
  bcols = n_b // _NUM_WORKERS
  assert bcols == _NBUF * _CHUNK
  nbh_c = _CHUNK // 128
  obw = _CHUNK * d
  jhw = nbh_c * 1024
  mesh = plsc.VectorSubcoreMesh(core_axis_name="c", subcore_axis_name="s")

  @functools.partial(
      pl.kernel,
      mesh=mesh,
      out_type=jax.ShapeDtypeStruct((n_s, n_jh, n_b // 128, 8, 128),
                                    jnp.float32),
      scratch_types=(
          [pltpu.VMEM((_CHUNK,), jnp.int32) for _ in range(2 * _NBUF)]
          + [pltpu.VMEM((_CHUNK, d), jnp.float32) for _ in range(2 * _NBUF)]
          + [pltpu.VMEM((n_jh, nbh_c, 8, 128), jnp.float32)
             for _ in range(_NBUF)]
          + [pltpu.SemaphoreType.DMA] * (4 * _NBUF + _NBUF)
      ),
      compiler_params=pltpu.CompilerParams(
          use_tc_tiling_on_sc=False, needs_layout_passes=False),
  )
  def emb_kernel(ids_hbm, table_hbm, out_hbm, *scr):
    idx_v = [scr[0:_NBUF], scr[_NBUF:2 * _NBUF]]
    rows_v = [scr[2 * _NBUF:3 * _NBUF], scr[3 * _NBUF:4 * _NBUF]]
    ob_v = scr[4 * _NBUF:5 * _NBUF]
    sem_i = [scr[5 * _NBUF:6 * _NBUF], scr[6 * _NBUF:7 * _NBUF]]
    sem_g = [scr[7 * _NBUF:8 * _NBUF], scr[8 * _NBUF:9 * _NBUF]]
    sem_o = scr[9 * _NBUF:10 * _NBUF]

    wid = lax.axis_index("s") * 2 + lax.axis_index("c")
    wb0 = wid * bcols
    wbh0 = wid * (bcols // 128)

    lane = jnp.arange(16, dtype=jnp.int32)
    i_jh = [(lane + j0) >> 3 for j0 in range(0, d, 16)]
    i_jl = [(lane + j0) & 7 for j0 in range(0, d, 16)]

    def idx_desc(s, p, b):
      return pltpu.make_async_copy(
          ids_hbm.at[s, pl.ds(wb0 + b * _CHUNK, _CHUNK)],
          idx_v[p][b], sem_i[p][b])

    def gather_descs(p, b):
      return [pltpu.make_async_copy(
          table_hbm.at[idx_v[p][b].at[pl.ds(g * _G, _G)]],
          rows_v[p][b].at[pl.ds(g * _G, _G)],
          sem_g[p][b]) for g in range(_CHUNK // _G)]

    def out_descs(s, b):
      return [pltpu.make_async_copy(
          ob_v[b],
          out_hbm.at[s, :, pl.ds(wbh0 + b * nbh_c, nbh_c)],
          sem_o[b])]

    def transpose_chunk(rows, ob):
      @plsc.parallel_loop(0, _CHUNK, unroll=8)
      def tok_body(tok):
        i_bh = jnp.broadcast_to(tok >> 7, (16,))
        i_bl = jnp.broadcast_to(tok & 127, (16,))
        for jj, j0 in enumerate(range(0, d, 16)):
          vals = rows[tok, pl.ds(j0, 16)]
          plsc.store_scatter(ob, [i_jh[jj], i_bh, i_jl[jj], i_bl], vals)

    for b in range(_NBUF):
      idx_desc(0, 0, b).start()
      idx_desc(1, 1, b).start()

    @pl.loop(0, n_s + 2, step=2)
    def round_body(r0):
      for dp in range(2):
        rr = r0 + dp
        p = dp
        @pl.when(rr < n_s)
        def _(rr=rr, p=p):
          for b in range(_NBUF):
            idx_desc(rr, p, b).wait()
            for dsc in gather_descs(p, b):
              dsc.start()
        @pl.when(jnp.logical_and(rr >= 1, rr <= n_s))
        def _(rr=rr, p=p):
          for b in range(_NBUF):
            for dsc in gather_descs(1 - p, b):
              dsc.wait()
            @pl.when(rr >= 2)
            def _(rr=rr, b=b):
              for dsc in out_descs(rr - 2, b):
                dsc.wait()
            transpose_chunk(rows_v[1 - p][b], ob_v[b])
            for dsc in out_descs(rr - 1, b):
              dsc.start()
            @pl.when(rr + 1 < n_s)
            def _(rr=rr, p=p, b=b):
              idx_desc(rr + 1, 1 - p, b).start()

    for b in range(_NBUF):
      for dsc in out_descs(n_s - 1, b):
        dsc.wait()

  return emb_kernel(ids_t, weights)


def kernel(token_ids, weights):
  bsz, seq = token_ids.shape
  d_model = weights.shape[1]
  ids_t = token_ids.T.astype(jnp.int32)
  out5 = _emb5(ids_t, weights, seq, bsz, d_model)
  return out5.transpose(2, 4, 0, 1, 3).reshape(bsz, seq, d_model)
